# Initial kernel scaffold; baseline (speedup 1.0000x reference)
import jax
import jax.numpy as jnp
from jax.experimental import pallas as pl


def kernel(x, edge_index, batch, W1_0, b1_0, W2_0, b2_0, W1_1, b1_1, W2_1, b2_1, W1_2, b1_2, W2_2, b2_2, cW1, cb1, bn_gamma, bn_beta, cW2, cb2):
    raise NotImplementedError("write your pallas kernel here")


# speedup vs baseline: 5.7780x; 5.7780x over previous
import functools

import jax
import jax.numpy as jnp
import numpy as np
from jax import lax
from jax.experimental import pallas as pl
from jax.experimental.pallas import tpu as pltpu
from jax.experimental.pallas import tpu_sc as plsc

N = 10000
E = 320000
D = 128
H = 128
G = 64
NC = 2

NCORES = 2
NSUB = 16
NW = NCORES * NSUB
EPW = E // NW
CH = 128
NFULL = EPW // CH
TAIL = EPW - NFULL * CH
RPT = 640

_sc_mesh = plsc.VectorSubcoreMesh(core_axis_name="c", subcore_axis_name="s")


@functools.partial(
    pl.kernel,
    out_type=jax.ShapeDtypeStruct((2 * N, H), jnp.float32),
    mesh=_sc_mesh,
    scratch_types=[
        pltpu.VMEM((CH,), jnp.int32),
        pltpu.VMEM((CH,), jnp.int32),
        pltpu.VMEM((CH, H), jnp.float32),
        pltpu.VMEM((TAIL,), jnp.int32),
        pltpu.VMEM((TAIL,), jnp.int32),
        pltpu.VMEM((TAIL, H), jnp.float32),
        pltpu.SemaphoreType.DMA,
        pltpu.VMEM_SHARED((N, H), jnp.float32),
    ],
)
def _sc_agg(h_hbm, src_hbm, dst_hbm, out_hbm,
            sidx, didx, rows, sidx2, didx2, rows2, gsem, agg_sh):
    cid = lax.axis_index("c")
    sid = lax.axis_index("s")
    wid = cid * NSUB + sid
    base = wid * EPW

    def _zrow(r, carry):
        for c8 in range(H // 16):
            rows[r, pl.ds(c8 * 16, 16)] = jnp.zeros((16,), jnp.float32)
        return carry
    lax.fori_loop(0, CH, _zrow, 0)
    row0 = sid * RPT

    @pl.when(sid < NSUB - 1)
    def _():
        for t in range(RPT // CH):
            pltpu.sync_copy(rows, agg_sh.at[pl.ds(row0 + t * CH, CH)])

    @pl.when(sid == NSUB - 1)
    def _():
        for t in range(3):
            pltpu.sync_copy(rows, agg_sh.at[pl.ds(row0 + t * CH, CH)])
        pltpu.sync_copy(rows.at[pl.ds(0, 16)],
                        agg_sh.at[pl.ds(row0 + 3 * CH, 16)])
    plsc.subcore_barrier()

    def _chunk(j, carry):
        off = base + j * CH
        pltpu.sync_copy(src_hbm.at[pl.ds(off, CH)], sidx)
        pltpu.sync_copy(dst_hbm.at[pl.ds(off, CH)], didx)
        pltpu.async_copy(h_hbm.at[sidx], rows, gsem).wait()
        pltpu.sync_copy(rows, agg_sh.at[didx], add=True)
        return carry
    lax.fori_loop(0, NFULL, _chunk, 0)

    if TAIL:
        off = base + NFULL * CH
        pltpu.sync_copy(src_hbm.at[pl.ds(off, TAIL)], sidx2)
        pltpu.sync_copy(dst_hbm.at[pl.ds(off, TAIL)], didx2)
        pltpu.async_copy(h_hbm.at[sidx2], rows2, gsem).wait()
        pltpu.sync_copy(rows2, agg_sh.at[didx2], add=True)

    plsc.subcore_barrier()

    @pl.when(sid < NSUB - 1)
    def _():
        pltpu.sync_copy(agg_sh.at[pl.ds(row0, RPT)],
                        out_hbm.at[pl.ds(cid * N + row0, RPT)])

    @pl.when(sid == NSUB - 1)
    def _():
        pltpu.sync_copy(agg_sh.at[pl.ds(row0, N - (NSUB - 1) * RPT)],
                        out_hbm.at[pl.ds(cid * N + row0, N - (NSUB - 1) * RPT)])


BR = 1000
NBLK = N // BR


def _mlp_body(h_ref, a0_ref, a1_ref, w1_ref, b1_ref, w2_ref, b2_ref, o_ref):
    z = h_ref[...] + a0_ref[...] + a1_ref[...]
    t = jnp.maximum(
        jnp.dot(z, w1_ref[...], preferred_element_type=jnp.float32) + b1_ref[...],
        0.0)
    o_ref[...] = jnp.maximum(
        jnp.dot(t, w2_ref[...], preferred_element_type=jnp.float32) + b2_ref[...],
        0.0)


_mlp = pl.pallas_call(
    _mlp_body,
    grid=(NBLK,),
    in_specs=[
        pl.BlockSpec((BR, H), lambda i: (i, 0)),
        pl.BlockSpec((BR, H), lambda i: (i, 0)),
        pl.BlockSpec((BR, H), lambda i: (NBLK + i, 0)),
        pl.BlockSpec((H, H), lambda i: (0, 0)),
        pl.BlockSpec((1, H), lambda i: (0, 0)),
        pl.BlockSpec((H, H), lambda i: (0, 0)),
        pl.BlockSpec((1, H), lambda i: (0, 0)),
    ],
    out_specs=pl.BlockSpec((BR, H), lambda i: (i, 0)),
    out_shape=jax.ShapeDtypeStruct((N, H), jnp.float32),
)

_BN_SCALE = float(1.0 / np.sqrt(1.0 + 1e-5))


def _pool_cls_body(b_ref, h1_ref, h2_ref, h3_ref, cw1_ref, cb1_ref,
                   g_ref, be_ref, cw2_ref, cb2_ref, o_ref, acc_ref):
    i = pl.program_id(0)

    @pl.when(i == 0)
    def _():
        acc_ref[...] = jnp.zeros_like(acc_ref)

    oh = (b_ref[...] == lax.broadcasted_iota(jnp.int32, (1, G), 1)
          ).astype(jnp.float32)
    hcat = jnp.concatenate([h1_ref[...], h2_ref[...], h3_ref[...]], axis=1)
    acc_ref[...] += jnp.dot(oh.T, hcat, preferred_element_type=jnp.float32)

    @pl.when(i == pl.num_programs(0) - 1)
    def _():
        z = jnp.dot(acc_ref[...], cw1_ref[...],
                    preferred_element_type=jnp.float32) + cb1_ref[...]
        z = z * _BN_SCALE * g_ref[...] + be_ref[...]
        z = jnp.maximum(z, 0.0)
        o_ref[...] = jnp.dot(z, cw2_ref[...],
                             preferred_element_type=jnp.float32) + cb2_ref[...]


_pool_cls = pl.pallas_call(
    _pool_cls_body,
    grid=(NBLK,),
    in_specs=[
        pl.BlockSpec((BR, 1), lambda i: (i, 0)),
        pl.BlockSpec((BR, H), lambda i: (i, 0)),
        pl.BlockSpec((BR, H), lambda i: (i, 0)),
        pl.BlockSpec((BR, H), lambda i: (i, 0)),
        pl.BlockSpec((3 * H, 2 * H), lambda i: (0, 0)),
        pl.BlockSpec((1, 2 * H), lambda i: (0, 0)),
        pl.BlockSpec((1, 2 * H), lambda i: (0, 0)),
        pl.BlockSpec((1, 2 * H), lambda i: (0, 0)),
        pl.BlockSpec((2 * H, 128), lambda i: (0, 0)),
        pl.BlockSpec((1, 128), lambda i: (0, 0)),
    ],
    out_specs=pl.BlockSpec((G, 128), lambda i: (0, 0)),
    out_shape=jax.ShapeDtypeStruct((G, 128), jnp.float32),
    scratch_shapes=[pltpu.VMEM((G, 3 * H), jnp.float32)],
)


def kernel(x, edge_index, batch, W1_0, b1_0, W2_0, b2_0, W1_1, b1_1, W2_1,
           b2_1, W1_2, b1_2, W2_2, b2_2, cW1, cb1, bn_gamma, bn_beta, cW2,
           cb2):
    src = edge_index[0]
    dst = edge_index[1]
    params = [(W1_0, b1_0, W2_0, b2_0), (W1_1, b1_1, W2_1, b2_1),
              (W1_2, b1_2, W2_2, b2_2)]

    h = x
    hs = []
    for (W1, b1, W2, b2) in params:
        agg = _sc_agg(h, src, dst)
        h = _mlp(h, agg, agg, W1, b1.reshape(1, H), W2, b2.reshape(1, H))
        hs.append(h)

    cW2p = jnp.zeros((2 * H, 128), jnp.float32).at[:, :NC].set(cW2)
    cb2p = jnp.zeros((1, 128), jnp.float32).at[0, :NC].set(cb2)
    out = _pool_cls(batch.reshape(N, 1), hs[0], hs[1], hs[2], cW1,
                    cb1.reshape(1, 2 * H), bn_gamma.reshape(1, 2 * H),
                    bn_beta.reshape(1, 2 * H), cW2p, cb2p)
    return out[:, :NC]
